# baseline (speedup 1.0000x reference)
import functools

import jax
import jax.numpy as jnp
from jax import lax
from jax.experimental import pallas as pl
from jax.experimental.pallas import tpu as pltpu
from jax.experimental.pallas import tpu_sc as plsc

NN = 50000
NE = 800000
DF = 128
DEA = 16
F = 64
DE = 16
NG = 64

NEP = 802816
CHUNK = 128
NCH = NEP // CHUNK
GCHUNK = 256
HALF = NN // 2
DUMMY = HALF
AGG_ROWS = 25600
ZROWS = AGG_ROWS // 16

EB = 2048
NEB = NEP // EB
XB = 1000
NXB = NN // XB

_SC_PARAMS = pltpu.CompilerParams(use_tc_tiling_on_sc=False)


@functools.cache
def _mesh():
    return plsc.VectorSubcoreMesh(core_axis_name="c", subcore_axis_name="s",
                                  num_cores=2, num_subcores=16)


def _proj_h_body(x_ref, w_ref, b_ref, o_ref):
    o_ref[...] = (jnp.dot(x_ref[...], w_ref[...],
                          preferred_element_type=jnp.float32) + b_ref[...])


def _proj_h(x, W, b):
    return pl.pallas_call(
        _proj_h_body,
        grid=(NXB,),
        in_specs=[pl.BlockSpec((XB, DF), lambda i: (i, 0)),
                  pl.BlockSpec((DF, F), lambda i: (0, 0)),
                  pl.BlockSpec((1, F), lambda i: (0, 0))],
        out_specs=pl.BlockSpec((XB, F), lambda i: (i, 0)),
        out_shape=jax.ShapeDtypeStruct((NN, F), jnp.float32),
    )(x, W, b.reshape(1, F))


def _proj_e(ea, W, b):
    return pl.pallas_call(
        _proj_h_body,
        grid=(NE // 1600,),
        in_specs=[pl.BlockSpec((1600, DEA), lambda i: (i, 0)),
                  pl.BlockSpec((DEA, DE), lambda i: (0, 0)),
                  pl.BlockSpec((1, DE), lambda i: (0, 0))],
        out_specs=pl.BlockSpec((1600, DE), lambda i: (i, 0)),
        out_shape=jax.ShapeDtypeStruct((NE, DE), jnp.float32),
    )(ea, W, b.reshape(1, DE))


def _route_body(d_ref, o_ref):
    d = d_ref[...]
    in0 = d < HALF
    dummy = DUMMY + (d & 511)
    o_ref[0, ...] = jnp.where(in0, d, dummy)
    o_ref[1, ...] = jnp.where(in0, dummy, d - HALF)


def _route(dst2):
    return pl.pallas_call(
        _route_body,
        in_specs=[pl.BlockSpec((1, NEP), lambda: (0, 0))],
        out_specs=pl.BlockSpec((2, 1, NEP), lambda: (0, 0, 0)),
        out_shape=jax.ShapeDtypeStruct((2, 1, NEP), jnp.int32),
    )(dst2)


def _edge_body(hd_ref, hs_ref, e_ref, wf_ref, ws_ref, bf_ref, bs_ref, m_ref):
    hd = hd_ref[...]
    hs = hs_ref[...]
    e = e_ref[...]
    wf = wf_ref[...]
    ws = ws_ref[...]
    zf = (jnp.dot(hd, wf[0:F], preferred_element_type=jnp.float32)
          + jnp.dot(hs, wf[F:2 * F], preferred_element_type=jnp.float32)
          + jnp.dot(e, wf[2 * F:2 * F + DE], preferred_element_type=jnp.float32)
          + bf_ref[...])
    zs = (jnp.dot(hd, ws[0:F], preferred_element_type=jnp.float32)
          + jnp.dot(hs, ws[F:2 * F], preferred_element_type=jnp.float32)
          + jnp.dot(e, ws[2 * F:2 * F + DE], preferred_element_type=jnp.float32)
          + bs_ref[...])
    m_ref[...] = jax.nn.sigmoid(zf) * jax.nn.softplus(zs)


def _edge(hd, hs, e, Wf, bf, Ws, bs):
    return pl.pallas_call(
        _edge_body,
        grid=(NEB,),
        in_specs=[pl.BlockSpec((EB, F), lambda i: (i, 0)),
                  pl.BlockSpec((EB, F), lambda i: (i, 0)),
                  pl.BlockSpec((EB, DE), lambda i: (i, 0)),
                  pl.BlockSpec((2 * F + DE, F), lambda i: (0, 0)),
                  pl.BlockSpec((2 * F + DE, F), lambda i: (0, 0)),
                  pl.BlockSpec((1, F), lambda i: (0, 0)),
                  pl.BlockSpec((1, F), lambda i: (0, 0))],
        out_specs=pl.BlockSpec((EB, F), lambda i: (i, 0)),
        out_shape=jax.ShapeDtypeStruct((NEP, F), jnp.float32),
    )(hd, hs, e, Wf, Ws, bf.reshape(1, F), bs.reshape(1, F))


def _combine_body(h_ref, a_ref, o_ref):
    o_ref[...] = jnp.tanh(h_ref[...] + a_ref[...])


def _combine(h, agg):
    return pl.pallas_call(
        _combine_body,
        grid=(NXB,),
        in_specs=[pl.BlockSpec((XB, F), lambda i: (i, 0)),
                  pl.BlockSpec((XB, F), lambda i: (i, 0))],
        out_specs=pl.BlockSpec((XB, F), lambda i: (i, 0)),
        out_shape=jax.ShapeDtypeStruct((NN, F), jnp.float32),
    )(h, agg)


def _pool_body(b_ref, h_ref, g_ref):
    @pl.when(pl.program_id(0) == 0)
    def _():
        g_ref[...] = jnp.zeros_like(g_ref)

    lab = b_ref[0, 0, :]
    oht = (lab[None, :]
           == lax.broadcasted_iota(jnp.int32, (NG, XB), 0)).astype(jnp.bfloat16)
    h = h_ref[...]
    h1 = h.astype(jnp.bfloat16)
    r1 = h - h1.astype(jnp.float32)
    h2 = r1.astype(jnp.bfloat16)
    r2 = r1 - h2.astype(jnp.float32)
    h3 = r2.astype(jnp.bfloat16)
    g_ref[...] += (jnp.dot(oht, h1, preferred_element_type=jnp.float32)
                   + jnp.dot(oht, h2, preferred_element_type=jnp.float32)
                   + jnp.dot(oht, h3, preferred_element_type=jnp.float32))


def _pool(batch3, h):
    return pl.pallas_call(
        _pool_body,
        grid=(NXB,),
        in_specs=[pl.BlockSpec((1, 1, XB), lambda i: (i, 0, 0)),
                  pl.BlockSpec((XB, F), lambda i: (i, 0))],
        out_specs=pl.BlockSpec((NG, F), lambda i: (0, 0)),
        out_shape=jax.ShapeDtypeStruct((NG, F), jnp.float32),
    )(batch3, h)


def _bn(h, gm, bt):
    mu = jnp.mean(h, axis=0, keepdims=True)
    var = jnp.mean((h - mu) ** 2, axis=0, keepdims=True)
    return gm * (h - mu) * lax.rsqrt(var + 1e-5) + bt


def _heads_body(*refs):
    g_ref = refs[0]
    out_ref = refs[-1]
    h = g_ref[...]
    for i in range(4):
        W, b, gm, bt = refs[1 + 4 * i:1 + 4 * i + 4]
        h = jnp.dot(h, W[...], preferred_element_type=jnp.float32) + b[...]
        h = jnp.maximum(_bn(h, gm[...], bt[...]), 0.0)
    tsW1, tsb1, tsg1, tsbe1, tsW2, tsb2, tsg2, tsbe2, tsWs, tsbs = refs[17:27]
    cols = []
    for t in range(5):
        y = jnp.dot(h, tsW1[t], preferred_element_type=jnp.float32) + tsb1[t]
        y = jnp.maximum(_bn(y, tsg1[t], tsbe1[t]), 0.0)
        y = jnp.dot(y, tsW2[t], preferred_element_type=jnp.float32) + tsb2[t]
        y = jnp.maximum(_bn(y, tsg2[t], tsbe2[t]), 0.0)
        w = tsWs[t]
        y = jnp.sum(y * w[:, 0][None, :], axis=1, keepdims=True) + tsbs[t]
        cols.append(1.0 / (1.0 + jnp.exp(-y)))
    out_ref[...] = jnp.concatenate(cols, axis=1)


def _heads(g, p):
    args = [g]
    for i in range(1, 5):
        dim = p['lin%d_b' % i].shape[0]
        args += [p['lin%d_W' % i], p['lin%d_b' % i].reshape(1, dim),
                 p['bn%d_g' % i].reshape(1, dim), p['bn%d_b' % i].reshape(1, dim)]
    args.append(jnp.stack([p['ts%d_W1' % t] for t in range(1, 6)]))
    args.append(jnp.stack([p['ts%d_b1' % t] for t in range(1, 6)])[:, None, :])
    args.append(jnp.stack([p['ts%d_g1' % t] for t in range(1, 6)])[:, None, :])
    args.append(jnp.stack([p['ts%d_be1' % t] for t in range(1, 6)])[:, None, :])
    args.append(jnp.stack([p['ts%d_W2' % t] for t in range(1, 6)]))
    args.append(jnp.stack([p['ts%d_b2' % t] for t in range(1, 6)])[:, None, :])
    args.append(jnp.stack([p['ts%d_g2' % t] for t in range(1, 6)])[:, None, :])
    args.append(jnp.stack([p['ts%d_be2' % t] for t in range(1, 6)])[:, None, :])
    args.append(jnp.stack([p['ts%d_Ws' % t] for t in range(1, 6)]))
    args.append(jnp.stack([p['ts%d_bs' % t] for t in range(1, 6)])[:, None, :])
    return pl.pallas_call(
        _heads_body,
        out_shape=jax.ShapeDtypeStruct((NG, 5), jnp.float32),
    )(*args)


def _gather(h, src2, dst2):
    @functools.partial(
        pl.kernel,
        out_type=(jax.ShapeDtypeStruct((NEP, F), jnp.float32),
                  jax.ShapeDtypeStruct((NEP, F), jnp.float32)),
        mesh=_mesh(),
        compiler_params=_SC_PARAMS)
    def k(h_hbm, src_hbm, dst_hbm, hd_hbm, hs_hbm):
        def body(src_v, dst_v, hd_v, hs_v):
            pltpu.sync_copy(h_hbm.at[dst_v.at[0]], hd_v)
            pltpu.sync_copy(h_hbm.at[src_v.at[0]], hs_v)

        pltpu.emit_pipeline(
            body,
            grid=(NEP // GCHUNK,),
            in_specs=[pl.BlockSpec((1, GCHUNK), lambda i: (0, i)),
                      pl.BlockSpec((1, GCHUNK), lambda i: (0, i))],
            out_specs=[pl.BlockSpec((GCHUNK, F), lambda i: (i, 0)),
                       pl.BlockSpec((GCHUNK, F), lambda i: (i, 0))],
            core_axis_name=("c", "s"),
            dimension_semantics=(pltpu.PARALLEL,),
        )(src_hbm, dst_hbm, hd_hbm, hs_hbm)

    return k(h, src2, dst2)


def _scatter(m, idx2, zeros):
    @functools.partial(
        pl.kernel,
        out_type=jax.ShapeDtypeStruct((NN, F), jnp.float32),
        mesh=_mesh(),
        scratch_types=[pltpu.VMEM_SHARED((AGG_ROWS, F), jnp.float32)],
        compiler_params=_SC_PARAMS)
    def k(m_hbm, idx_hbm, z_hbm, agg_hbm, agg_sh):
        c = lax.axis_index("c")
        s = lax.axis_index("s")
        pltpu.sync_copy(z_hbm.at[pl.ds(s * ZROWS, ZROWS)],
                        agg_sh.at[pl.ds(s * ZROWS, ZROWS)])
        plsc.subcore_barrier()

        my_idx = idx_hbm.at[c]

        def body(m_v, i_v):
            pltpu.sync_copy(m_v, agg_sh.at[i_v.at[0]], add=True)

        pltpu.emit_pipeline(
            body,
            grid=(NCH,),
            in_specs=[pl.BlockSpec((CHUNK, F), lambda i: (i, 0)),
                      pl.BlockSpec((1, CHUNK), lambda i: (0, i))],
            out_specs=[],
            core_axis_name="s",
            dimension_semantics=(pltpu.PARALLEL,),
        )(m_hbm, my_idx)
        plsc.subcore_barrier()

        @pl.loop(s, 125, step=16)
        def _(ch):
            off = ch * 200
            pltpu.sync_copy(agg_sh.at[pl.ds(off, 200)],
                            agg_hbm.at[pl.ds(c * HALF + off, 200)])

    return k(m, idx2, zeros)


@jax.jit
def _run(x, edge_index, edge_attr, batch, params):
    p = params
    padn = NEP - NE
    pad0 = jnp.zeros((padn,), jnp.int32)
    src2 = jnp.concatenate([edge_index[0], pad0]).reshape(1, NEP)
    dst2 = jnp.concatenate([edge_index[1], pad0]).reshape(1, NEP)
    dstr = jnp.concatenate([edge_index[1],
                            jnp.full((padn,), NN, jnp.int32)]).reshape(1, NEP)
    h = _proj_h(x, p['gnn_W1'], p['gnn_b1'])
    e = jnp.concatenate([_proj_e(edge_attr, p['gnn_W2'], p['gnn_b2']),
                         jnp.zeros((padn, DE), jnp.float32)])
    idx2 = _route(dstr)
    zeros = jnp.zeros((AGG_ROWS, F), jnp.float32)

    for cname in ('conv1', 'conv2'):
        hd, hs = _gather(h, src2, dst2)
        m = _edge(hd, hs, e, p[cname + '_Wf'], p[cname + '_bf'],
                  p[cname + '_Ws'], p[cname + '_bs'])
        agg = _scatter(m, idx2, zeros)
        h = _combine(h, agg)

    g = jax.ops.segment_sum(h, batch, num_segments=NG)
    y = _heads(g, p)
    return tuple(y[:, t:t + 1] for t in range(5))


def kernel(x, edge_index, edge_attr, batch, params):
    return _run(x, edge_index, edge_attr, batch, params)


# speedup vs baseline: 1.2388x; 1.2388x over previous
import functools

import jax
import jax.numpy as jnp
from jax import lax
from jax.experimental import pallas as pl
from jax.experimental.pallas import tpu as pltpu
from jax.experimental.pallas import tpu_sc as plsc

NN = 50000
NE = 800000
DF = 128
DEA = 16
F = 64
DE = 16
NG = 64

NEP = 802816
CHUNK = 128
NCH = NEP // CHUNK
GCHUNK = 128
HALF = NN // 2
DUMMY = HALF
AGG_ROWS = 25600
ZROWS = AGG_ROWS // 16

EB = 2048
NEB = NEP // EB
XB = 1000
NXB = NN // XB

_SC_PARAMS = pltpu.CompilerParams(use_tc_tiling_on_sc=False)


@functools.cache
def _mesh():
    return plsc.VectorSubcoreMesh(core_axis_name="c", subcore_axis_name="s",
                                  num_cores=2, num_subcores=16)


def _proj_h_body(x_ref, w_ref, b_ref, o_ref):
    o_ref[...] = (jnp.dot(x_ref[...], w_ref[...],
                          preferred_element_type=jnp.float32) + b_ref[...])


def _proj_h(x, W, b):
    return pl.pallas_call(
        _proj_h_body,
        grid=(NXB,),
        in_specs=[pl.BlockSpec((XB, DF), lambda i: (i, 0)),
                  pl.BlockSpec((DF, F), lambda i: (0, 0)),
                  pl.BlockSpec((1, F), lambda i: (0, 0))],
        out_specs=pl.BlockSpec((XB, F), lambda i: (i, 0)),
        out_shape=jax.ShapeDtypeStruct((NN, F), jnp.float32),
    )(x, W, b.reshape(1, F))


def _proj_e(ea, W, b):
    return pl.pallas_call(
        _proj_h_body,
        grid=(NE // 1600,),
        in_specs=[pl.BlockSpec((1600, DEA), lambda i: (i, 0)),
                  pl.BlockSpec((DEA, DE), lambda i: (0, 0)),
                  pl.BlockSpec((1, DE), lambda i: (0, 0))],
        out_specs=pl.BlockSpec((1600, DE), lambda i: (i, 0)),
        out_shape=jax.ShapeDtypeStruct((NE, DE), jnp.float32),
    )(ea, W, b.reshape(1, DE))


def _route_body(d_ref, o_ref):
    d = d_ref[...]
    in0 = d < HALF
    dummy = DUMMY + (d & 511)
    o_ref[0, ...] = jnp.where(in0, d, dummy)
    o_ref[1, ...] = jnp.where(in0, dummy, d - HALF)


def _route(dst2):
    return pl.pallas_call(
        _route_body,
        in_specs=[pl.BlockSpec((1, NEP), lambda: (0, 0))],
        out_specs=pl.BlockSpec((2, 1, NEP), lambda: (0, 0, 0)),
        out_shape=jax.ShapeDtypeStruct((2, 1, NEP), jnp.int32),
    )(dst2)


def _edge_body(hd_ref, hs_ref, e_ref, wf_ref, ws_ref, bf_ref, bs_ref, m_ref):
    hd = hd_ref[...]
    hs = hs_ref[...]
    e = e_ref[...]
    wf = wf_ref[...]
    ws = ws_ref[...]
    zf = (jnp.dot(hd, wf[0:F], preferred_element_type=jnp.float32)
          + jnp.dot(hs, wf[F:2 * F], preferred_element_type=jnp.float32)
          + jnp.dot(e, wf[2 * F:2 * F + DE], preferred_element_type=jnp.float32)
          + bf_ref[...])
    zs = (jnp.dot(hd, ws[0:F], preferred_element_type=jnp.float32)
          + jnp.dot(hs, ws[F:2 * F], preferred_element_type=jnp.float32)
          + jnp.dot(e, ws[2 * F:2 * F + DE], preferred_element_type=jnp.float32)
          + bs_ref[...])
    m_ref[...] = jax.nn.sigmoid(zf) * jax.nn.softplus(zs)


def _edge(hd, hs, e, Wf, bf, Ws, bs):
    return pl.pallas_call(
        _edge_body,
        grid=(NEB,),
        in_specs=[pl.BlockSpec((EB, F), lambda i: (i, 0)),
                  pl.BlockSpec((EB, F), lambda i: (i, 0)),
                  pl.BlockSpec((EB, DE), lambda i: (i, 0)),
                  pl.BlockSpec((2 * F + DE, F), lambda i: (0, 0)),
                  pl.BlockSpec((2 * F + DE, F), lambda i: (0, 0)),
                  pl.BlockSpec((1, F), lambda i: (0, 0)),
                  pl.BlockSpec((1, F), lambda i: (0, 0))],
        out_specs=pl.BlockSpec((EB, F), lambda i: (i, 0)),
        out_shape=jax.ShapeDtypeStruct((NEP, F), jnp.float32),
    )(hd, hs, e, Wf, Ws, bf.reshape(1, F), bs.reshape(1, F))


def _combine_body(h_ref, a_ref, o_ref):
    o_ref[...] = jnp.tanh(h_ref[...] + a_ref[...])


def _combine(h, agg):
    return pl.pallas_call(
        _combine_body,
        grid=(NXB,),
        in_specs=[pl.BlockSpec((XB, F), lambda i: (i, 0)),
                  pl.BlockSpec((XB, F), lambda i: (i, 0))],
        out_specs=pl.BlockSpec((XB, F), lambda i: (i, 0)),
        out_shape=jax.ShapeDtypeStruct((NN, F), jnp.float32),
    )(h, agg)


def _pool_body(b_ref, h_ref, g_ref):
    @pl.when(pl.program_id(0) == 0)
    def _():
        g_ref[...] = jnp.zeros_like(g_ref)

    lab = b_ref[0, 0, :]
    oht = (lab[None, :]
           == lax.broadcasted_iota(jnp.int32, (NG, XB), 0)).astype(jnp.bfloat16)
    h = h_ref[...]
    h1 = h.astype(jnp.bfloat16)
    r1 = h - h1.astype(jnp.float32)
    h2 = r1.astype(jnp.bfloat16)
    r2 = r1 - h2.astype(jnp.float32)
    h3 = r2.astype(jnp.bfloat16)
    g_ref[...] += (jnp.dot(oht, h1, preferred_element_type=jnp.float32)
                   + jnp.dot(oht, h2, preferred_element_type=jnp.float32)
                   + jnp.dot(oht, h3, preferred_element_type=jnp.float32))


def _pool(batch3, h):
    return pl.pallas_call(
        _pool_body,
        grid=(NXB,),
        in_specs=[pl.BlockSpec((1, 1, XB), lambda i: (i, 0, 0)),
                  pl.BlockSpec((XB, F), lambda i: (i, 0))],
        out_specs=pl.BlockSpec((NG, F), lambda i: (0, 0)),
        out_shape=jax.ShapeDtypeStruct((NG, F), jnp.float32),
    )(batch3, h)


def _bn(h, gm, bt):
    mu = jnp.mean(h, axis=0, keepdims=True)
    var = jnp.mean((h - mu) ** 2, axis=0, keepdims=True)
    return gm * (h - mu) * lax.rsqrt(var + 1e-5) + bt


def _heads_body(*refs):
    g_ref = refs[0]
    out_ref = refs[-1]
    h = g_ref[...]
    for i in range(4):
        W, b, gm, bt = refs[1 + 4 * i:1 + 4 * i + 4]
        h = jnp.dot(h, W[...], preferred_element_type=jnp.float32) + b[...]
        h = jnp.maximum(_bn(h, gm[...], bt[...]), 0.0)
    tsW1, tsb1, tsg1, tsbe1, tsW2, tsb2, tsg2, tsbe2, tsWs, tsbs = refs[17:27]
    cols = []
    for t in range(5):
        y = jnp.dot(h, tsW1[t], preferred_element_type=jnp.float32) + tsb1[t]
        y = jnp.maximum(_bn(y, tsg1[t], tsbe1[t]), 0.0)
        y = jnp.dot(y, tsW2[t], preferred_element_type=jnp.float32) + tsb2[t]
        y = jnp.maximum(_bn(y, tsg2[t], tsbe2[t]), 0.0)
        w = tsWs[t]
        y = jnp.sum(y * w[:, 0][None, :], axis=1, keepdims=True) + tsbs[t]
        cols.append(1.0 / (1.0 + jnp.exp(-y)))
    out_ref[...] = jnp.concatenate(cols, axis=1)


def _heads(g, p):
    args = [g]
    for i in range(1, 5):
        dim = p['lin%d_b' % i].shape[0]
        args += [p['lin%d_W' % i], p['lin%d_b' % i].reshape(1, dim),
                 p['bn%d_g' % i].reshape(1, dim), p['bn%d_b' % i].reshape(1, dim)]
    args.append(jnp.stack([p['ts%d_W1' % t] for t in range(1, 6)]))
    args.append(jnp.stack([p['ts%d_b1' % t] for t in range(1, 6)])[:, None, :])
    args.append(jnp.stack([p['ts%d_g1' % t] for t in range(1, 6)])[:, None, :])
    args.append(jnp.stack([p['ts%d_be1' % t] for t in range(1, 6)])[:, None, :])
    args.append(jnp.stack([p['ts%d_W2' % t] for t in range(1, 6)]))
    args.append(jnp.stack([p['ts%d_b2' % t] for t in range(1, 6)])[:, None, :])
    args.append(jnp.stack([p['ts%d_g2' % t] for t in range(1, 6)])[:, None, :])
    args.append(jnp.stack([p['ts%d_be2' % t] for t in range(1, 6)])[:, None, :])
    args.append(jnp.stack([p['ts%d_Ws' % t] for t in range(1, 6)]))
    args.append(jnp.stack([p['ts%d_bs' % t] for t in range(1, 6)])[:, None, :])
    return pl.pallas_call(
        _heads_body,
        out_shape=jax.ShapeDtypeStruct((NG, 5), jnp.float32),
    )(*args)


def _gather(h, src2, dst2):
    @functools.partial(
        pl.kernel,
        out_type=(jax.ShapeDtypeStruct((NEP, F), jnp.float32),
                  jax.ShapeDtypeStruct((NEP, F), jnp.float32)),
        mesh=_mesh(),
        compiler_params=_SC_PARAMS)
    def k(h_hbm, src_hbm, dst_hbm, hd_hbm, hs_hbm):
        def body(src_v, dst_v, hd_v, hs_v):
            pltpu.sync_copy(h_hbm.at[dst_v.at[0]], hd_v)
            pltpu.sync_copy(h_hbm.at[src_v.at[0]], hs_v)

        pltpu.emit_pipeline(
            body,
            grid=(NEP // GCHUNK,),
            in_specs=[pl.BlockSpec((1, GCHUNK), lambda i: (0, i)),
                      pl.BlockSpec((1, GCHUNK), lambda i: (0, i))],
            out_specs=[pl.BlockSpec((GCHUNK, F), lambda i: (i, 0)),
                       pl.BlockSpec((GCHUNK, F), lambda i: (i, 0))],
            core_axis_name=("c", "s"),
            dimension_semantics=(pltpu.PARALLEL,),
        )(src_hbm, dst_hbm, hd_hbm, hs_hbm)

    return k(h, src2, dst2)


def _scatter(m, idx2, zeros):
    @functools.partial(
        pl.kernel,
        out_type=jax.ShapeDtypeStruct((NN, F), jnp.float32),
        mesh=_mesh(),
        scratch_types=[pltpu.VMEM_SHARED((AGG_ROWS, F), jnp.float32)],
        compiler_params=_SC_PARAMS)
    def k(m_hbm, idx_hbm, z_hbm, agg_hbm, agg_sh):
        c = lax.axis_index("c")
        s = lax.axis_index("s")
        pltpu.sync_copy(z_hbm.at[pl.ds(s * ZROWS, ZROWS)],
                        agg_sh.at[pl.ds(s * ZROWS, ZROWS)])
        plsc.subcore_barrier()

        my_idx = idx_hbm.at[c]

        def body(m_v, i_v):
            pltpu.sync_copy(m_v, agg_sh.at[i_v.at[0]], add=True)

        pltpu.emit_pipeline(
            body,
            grid=(NCH,),
            in_specs=[pl.BlockSpec((CHUNK, F), lambda i: (i, 0)),
                      pl.BlockSpec((1, CHUNK), lambda i: (0, i))],
            out_specs=[],
            core_axis_name="s",
            dimension_semantics=(pltpu.PARALLEL,),
        )(m_hbm, my_idx)
        plsc.subcore_barrier()

        @pl.loop(s, 125, step=16)
        def _(ch):
            off = ch * 200
            pltpu.sync_copy(agg_sh.at[pl.ds(off, 200)],
                            agg_hbm.at[pl.ds(c * HALF + off, 200)])

    return k(m, idx2, zeros)


@jax.jit
def _run(x, edge_index, edge_attr, batch, params):
    p = params
    padn = NEP - NE
    pad0 = jnp.zeros((padn,), jnp.int32)
    src2 = jnp.concatenate([edge_index[0], pad0]).reshape(1, NEP)
    dst2 = jnp.concatenate([edge_index[1], pad0]).reshape(1, NEP)
    dstr = jnp.concatenate([edge_index[1],
                            jnp.full((padn,), NN, jnp.int32)]).reshape(1, NEP)
    h = _proj_h(x, p['gnn_W1'], p['gnn_b1'])
    e = jnp.concatenate([_proj_e(edge_attr, p['gnn_W2'], p['gnn_b2']),
                         jnp.zeros((padn, DE), jnp.float32)])
    idx2 = _route(dstr)
    zeros = jnp.zeros((AGG_ROWS, F), jnp.float32)

    for cname in ('conv1', 'conv2'):
        hd, hs = _gather(h, src2, dst2)
        m = _edge(hd, hs, e, p[cname + '_Wf'], p[cname + '_bf'],
                  p[cname + '_Ws'], p[cname + '_bs'])
        agg = _scatter(m, idx2, zeros)
        h = _combine(h, agg)

    g = jax.ops.segment_sum(h, batch, num_segments=NG)
    y = _heads(g, p)
    return tuple(y[:, t:t + 1] for t in range(5))


def kernel(x, edge_index, edge_attr, batch, params):
    return _run(x, edge_index, edge_attr, batch, params)
